# baseline (speedup 1.0000x reference)
import jax
import jax.numpy as jnp
from jax.experimental import pallas as pl
from jax.experimental.pallas import tpu as pltpu

_DN = (((1,), (0,)), ((), ()))


def _make_body(NI, NK, BN, BK):
    def _body(f_ref, w1_ref, b1_ref, w2_ref, b2_ref, wh_ref, bh_ref,
              out_ref, acc_ref):
        k = pl.program_id(0)
        i = pl.program_id(1)
        rows = pl.ds(i * BN, BN)

        part = jnp.dot(f_ref[...], w1_ref[...],
                       preferred_element_type=jnp.float32)

        @pl.when(k == 0)
        def _init():
            acc_ref[rows, :] = part + b1_ref[...]

        @pl.when(k > 0)
        def _accum():
            acc_new = acc_ref[rows, :] + part
            acc_ref[rows, :] = acc_new

            @pl.when(k == NK - 1)
            def _finish():
                x = jnp.maximum(acc_new, 0.0)
                x = jax.lax.dot_general(x, w2_ref[...], _DN,
                                        preferred_element_type=jnp.float32)
                x = jnp.maximum(x + b2_ref[...], 0.0)
                y = jax.lax.dot_general(x, wh_ref[...], _DN,
                                        preferred_element_type=jnp.float32)
                out_ref[...] = y + bh_ref[...]

    return _body


def kernel(feature_vectors, W1, b1, W2, b2, Wc, bc, Wr, br):
    N, D = feature_vectors.shape
    H = W1.shape[1]
    NC = Wc.shape[1]
    NR = Wr.shape[1]

    BN = 1000
    BK = 1792
    assert N % BN == 0 and D % BK == 0
    NI = N // BN
    NK = D // BK
    grid = (NK, NI)

    Wh = jnp.concatenate([Wc, Wr], axis=1).astype(jnp.bfloat16)
    W2b = W2.astype(jnp.bfloat16)
    bh = jnp.concatenate([bc, br])[None, :]
    b1_2d = b1[None, :]
    b2_2d = b2[None, :]

    out = pl.pallas_call(
        _make_body(NI, NK, BN, BK),
        grid=grid,
        in_specs=[
            pl.BlockSpec((BN, BK), lambda k, i: (i, k)),
            pl.BlockSpec((BK, H), lambda k, i: (k, 0)),
            pl.BlockSpec((1, H), lambda k, i: (0, 0)),
            pl.BlockSpec((H, H), lambda k, i: (0, 0)),
            pl.BlockSpec((1, H), lambda k, i: (0, 0)),
            pl.BlockSpec((H, NC + NR), lambda k, i: (0, 0)),
            pl.BlockSpec((1, NC + NR), lambda k, i: (0, 0)),
        ],
        out_specs=pl.BlockSpec((BN, NC + NR), lambda k, i: (i, 0)),
        out_shape=jax.ShapeDtypeStruct((N, NC + NR), jnp.float32),
        scratch_shapes=[
            pltpu.VMEM((N, H), jnp.float32),
        ],
        compiler_params=pltpu.CompilerParams(
            dimension_semantics=("arbitrary", "arbitrary"),
            vmem_limit_bytes=67_000_000,
        ),
    )(feature_vectors, W1, b1_2d, W2b, b2_2d, Wh, bh)
    return out[:, :NC], out[:, NC:]


# speedup vs baseline: 1.1027x; 1.1027x over previous
import jax
import jax.numpy as jnp
from jax.experimental import pallas as pl
from jax.experimental.pallas import tpu as pltpu

_DN = (((1,), (0,)), ((), ()))


def _make_body(NI, NK, BN, BK):
    def _body(f_ref, w1_hbm, b1_ref, w2_ref, b2_ref, wh_ref, bh_ref,
              out_ref, acc_ref, w1buf_ref, sems):
        k = pl.program_id(0)
        i = pl.program_id(1)
        rows = pl.ds(i * BN, BN)

        def _slab_copy(kk):
            return pltpu.make_async_copy(
                w1_hbm.at[pl.ds(kk * BK, BK), :],
                w1buf_ref.at[kk % 2],
                sems.at[kk % 2],
            )

        @pl.when((k == 0) & (i == 0))
        def _first():
            _slab_copy(0).start()

        @pl.when(i == 0)
        def _rotate():
            _slab_copy(k).wait()

            @pl.when(k + 1 < NK)
            def _kick():
                _slab_copy(k + 1).start()

        part = jax.lax.dot_general(f_ref[...], w1buf_ref[k % 2], _DN,
                                   preferred_element_type=jnp.float32)
        prev = acc_ref[rows, :]
        acc_new = jnp.where(k > 0, prev + part, part)
        acc_ref[rows, :] = acc_new

        @pl.when(k == NK - 1)
        def _finish():
            x = jnp.maximum(acc_new + b1_ref[...], 0.0)
            x = jax.lax.dot_general(x, w2_ref[...], _DN,
                                    preferred_element_type=jnp.float32)
            x = jnp.maximum(x + b2_ref[...], 0.0)
            y = jax.lax.dot_general(x, wh_ref[...], _DN,
                                    preferred_element_type=jnp.float32)
            out_ref[...] = y + bh_ref[...]

    return _body


def kernel(feature_vectors, W1, b1, W2, b2, Wc, bc, Wr, br):
    N, D = feature_vectors.shape
    H = W1.shape[1]
    NC = Wc.shape[1]
    NR = Wr.shape[1]

    BN = 1000
    BK = 1792
    assert N % BN == 0 and D % BK == 0
    NI = N // BN
    NK = D // BK
    grid = (NK, NI)

    Wh = jnp.concatenate([Wc, Wr], axis=1).astype(jnp.bfloat16)
    W2b = W2.astype(jnp.bfloat16)
    bh = jnp.concatenate([bc, br])[None, :]
    b1_2d = b1[None, :]
    b2_2d = b2[None, :]

    out = pl.pallas_call(
        _make_body(NI, NK, BN, BK),
        grid=grid,
        in_specs=[
            pl.BlockSpec((BN, BK), lambda k, i: (i, k)),
            pl.BlockSpec(memory_space=pl.ANY),
            pl.BlockSpec((1, H), lambda k, i: (0, 0)),
            pl.BlockSpec((H, H), lambda k, i: (0, 0)),
            pl.BlockSpec((1, H), lambda k, i: (0, 0)),
            pl.BlockSpec((H, NC + NR), lambda k, i: (0, 0)),
            pl.BlockSpec((1, NC + NR), lambda k, i: (0, 0)),
        ],
        out_specs=pl.BlockSpec((BN, NC + NR), lambda k, i: (i, 0)),
        out_shape=jax.ShapeDtypeStruct((N, NC + NR), jnp.float32),
        scratch_shapes=[
            pltpu.VMEM((N, H), jnp.float32),
            pltpu.VMEM((2, BK, H), jnp.float32),
            pltpu.SemaphoreType.DMA((2,)),
        ],
        compiler_params=pltpu.CompilerParams(
            dimension_semantics=("arbitrary", "arbitrary"),
            vmem_limit_bytes=63 * 1024 * 1024,
        ),
    )(feature_vectors, W1, b1_2d, W2b, b2_2d, Wh, bh)
    return out[:, :NC], out[:, NC:]


# speedup vs baseline: 1.1099x; 1.0065x over previous
import jax
import jax.numpy as jnp
from jax.experimental import pallas as pl
from jax.experimental.pallas import tpu as pltpu

_DN = (((1,), (0,)), ((), ()))


def _make_body(NI, NK, BN, BK):
    def _body(f_ref, w1_hbm, b1_ref, w2_ref, b2_ref, wc_ref, bc_ref,
              wr_ref, br_ref, outc_ref, outr_ref,
              acc_ref, w1buf_ref, sems):
        k = pl.program_id(0)
        i = pl.program_id(1)
        rows = pl.ds(i * BN, BN)

        def _slab_copy(kk):
            return pltpu.make_async_copy(
                w1_hbm.at[pl.ds(kk * BK, BK), :],
                w1buf_ref.at[kk % 2],
                sems.at[kk % 2],
            )

        @pl.when((k == 0) & (i == 0))
        def _first():
            _slab_copy(0).start()

        @pl.when(i == 0)
        def _rotate():
            _slab_copy(k).wait()

            @pl.when(k + 1 < NK)
            def _kick():
                _slab_copy(k + 1).start()

        part = jax.lax.dot_general(f_ref[...], w1buf_ref[k % 2], _DN,
                                   preferred_element_type=jnp.float32)
        prev = acc_ref[rows, :]
        acc_new = jnp.where(k > 0, prev + part, part)
        acc_ref[rows, :] = acc_new

        @pl.when(k == NK - 1)
        def _finish():
            x = jnp.maximum(acc_new + b1_ref[...], 0.0)
            x = jax.lax.dot_general(x, w2_ref[...], _DN,
                                    preferred_element_type=jnp.float32)
            x = jnp.maximum(x + b2_ref[...], 0.0)
            yc = jax.lax.dot_general(x, wc_ref[...], _DN,
                                     preferred_element_type=jnp.float32)
            outc_ref[...] = yc + bc_ref[...]
            yr = jax.lax.dot_general(x, wr_ref[...], _DN,
                                     preferred_element_type=jnp.float32)
            outr_ref[...] = yr + br_ref[...]

    return _body


def kernel(feature_vectors, W1, b1, W2, b2, Wc, bc, Wr, br):
    N, D = feature_vectors.shape
    H = W1.shape[1]
    NC = Wc.shape[1]
    NR = Wr.shape[1]

    BN = 1000
    BK = 1792
    assert N % BN == 0 and D % BK == 0
    NI = N // BN
    NK = D // BK
    grid = (NK, NI)

    outc, outr = pl.pallas_call(
        _make_body(NI, NK, BN, BK),
        grid=grid,
        in_specs=[
            pl.BlockSpec((BN, BK), lambda k, i: (i, k)),
            pl.BlockSpec(memory_space=pl.ANY),
            pl.BlockSpec((1, H), lambda k, i: (0, 0)),
            pl.BlockSpec((H, H), lambda k, i: (0, 0)),
            pl.BlockSpec((1, H), lambda k, i: (0, 0)),
            pl.BlockSpec((H, NC), lambda k, i: (0, 0)),
            pl.BlockSpec((1, NC), lambda k, i: (0, 0)),
            pl.BlockSpec((H, NR), lambda k, i: (0, 0)),
            pl.BlockSpec((1, NR), lambda k, i: (0, 0)),
        ],
        out_specs=[
            pl.BlockSpec((BN, NC), lambda k, i: (i, 0)),
            pl.BlockSpec((BN, NR), lambda k, i: (i, 0)),
        ],
        out_shape=[
            jax.ShapeDtypeStruct((N, NC), jnp.float32),
            jax.ShapeDtypeStruct((N, NR), jnp.float32),
        ],
        scratch_shapes=[
            pltpu.VMEM((N, H), jnp.float32),
            pltpu.VMEM((2, BK, H), jnp.float32),
            pltpu.SemaphoreType.DMA((2,)),
        ],
        compiler_params=pltpu.CompilerParams(
            dimension_semantics=("arbitrary", "arbitrary"),
            vmem_limit_bytes=66_900_000,
        ),
    )(feature_vectors, W1, b1[None, :], W2, b2[None, :],
      Wc, bc[None, :], Wr, br[None, :])
    return outc, outr
